# Initial kernel scaffold; baseline (speedup 1.0000x reference)
import jax
import jax.numpy as jnp
from jax.experimental import pallas as pl


def kernel(features, xyz_0, xyz_1, xyz_2, xyz_3, neigh_idx_0, neigh_idx_1, neigh_idx_2, neigh_idx_3, sub_idx_0, sub_idx_1, sub_idx_2, sub_idx_3, interp_idx_0, interp_idx_1, interp_idx_2, interp_idx_3, params):
    raise NotImplementedError("write your pallas kernel here")


# speedup vs baseline: 5.4130x; 5.4130x over previous
import functools

import jax
import jax.numpy as jnp
from jax import lax
from jax.experimental import pallas as pl
from jax.experimental.pallas import tpu as pltpu
from jax.experimental.pallas import tpu_sc as plsc

NUM_LAYERS_ = 4
D_OUT_ = [16, 64, 128, 256]
NS_ = [40960, 10240, 2560, 640, 160]
K_ = 16
B_ = 2

_NW = 32
_IDXROW = 128


def _pad16(c):
    return (c + 15) // 16 * 16


def _pallas_call(*args, **kwargs):
    return pl.pallas_call(*args, **kwargs)


def _lrelu(x):
    return jnp.where(x >= 0, x, 0.2 * x)


def _dot(a, b):
    return jax.lax.dot_general(
        a, b, (((a.ndim - 1,), (0,)), ((), ())),
        preferred_element_type=jnp.float32,
        precision=jax.lax.Precision.HIGHEST)


@functools.lru_cache(maxsize=None)
def _make_sc_gather(V, D, Mp):
    per_w = Mp // _NW
    nrow = per_w // _IDXROW
    nsub = 1
    for cand in (8, 4, 2):
        if nrow % cand == 0 and cand * _IDXROW * D * 4 <= 262144:
            nsub = cand
            break
    ch = nsub * _IDXROW
    n_iter = nrow // nsub
    mesh = plsc.VectorSubcoreMesh(core_axis_name="c", subcore_axis_name="s")

    @functools.partial(
        pl.kernel, mesh=mesh,
        compiler_params=pltpu.CompilerParams(use_tc_tiling_on_sc=False),
        out_type=jax.ShapeDtypeStruct((Mp, D), jnp.float32),
        scratch_types=[
            pltpu.VMEM((nsub, _IDXROW), jnp.int32),
            pltpu.VMEM((ch, D), jnp.float32),
            pltpu.SemaphoreType.DMA,
        ])
    def k(table_hbm, idx_hbm, out_hbm, idx_v, rows_v, sem):
        wid = lax.axis_index("s") * 2 + lax.axis_index("c")
        ibase = wid * nrow

        def body(c, carry):
            irow = ibase + c * nsub
            pltpu.sync_copy(idx_hbm.at[pl.ds(irow, nsub)], idx_v)
            cps = [
                pltpu.async_copy(table_hbm.at[idx_v.at[j]],
                                 rows_v.at[pl.ds(j * _IDXROW, _IDXROW)], sem)
                for j in range(nsub)
            ]
            for cp in cps:
                cp.wait()
            pltpu.sync_copy(rows_v, out_hbm.at[pl.ds(irow * _IDXROW, ch)])
            return carry

        lax.fori_loop(0, n_iter, body, 0)

    return k


def _sc_gather(table, idx_flat):
    V, D = table.shape
    M = idx_flat.shape[0]
    Mp = (M + 4095) // 4096 * 4096
    if Mp != M:
        idx_flat = jnp.concatenate(
            [idx_flat, jnp.zeros((Mp - M,), jnp.int32)])
    idx2d = idx_flat.reshape(Mp // _IDXROW, _IDXROW)
    out = _make_sc_gather(V, D, Mp)(table, idx2d)
    return out[:M] if Mp != M else out


def _flat_idx(idx, n_table):
    b = idx.shape[0]
    off = (jnp.arange(b, dtype=jnp.int32) * n_table)[:, None, None]
    return (idx.astype(jnp.int32) + off).reshape(-1)


def _full_spec(shape):
    return pl.BlockSpec(shape, lambda i: tuple(0 for _ in shape))


def _row_spec(tn, c):
    return pl.BlockSpec((tn, c), lambda i: (i, 0))


def _mlp_chain_kernel(x, Ws, bs, acts, tn):
    R = x.shape[0]
    cout = Ws[-1].shape[1]

    def body(x_ref, *refs):
        n = len(Ws)
        o_ref = refs[2 * n]
        y = x_ref[...]
        for t in range(n):
            y = _dot(y, refs[2 * t][...]) + refs[2 * t + 1][...]
            if acts[t]:
                y = _lrelu(y)
        o_ref[...] = y

    ins = [x]
    specs = [_row_spec(tn, x.shape[1])]
    for W, b in zip(Ws, bs):
        ins += [W, b.reshape(1, -1)]
        specs += [_full_spec(W.shape), _full_spec((1, b.shape[0]))]
    return _pallas_call(
        body, grid=(R // tn,),
        in_specs=specs,
        out_specs=_row_spec(tn, cout),
        out_shape=jax.ShapeDtypeStruct((R, cout), jnp.float32),
    )(*ins)


def _block_a(x, xyz, p, din, dout, tn):
    d2 = dout // 2
    w1 = _pad16(3 + d2)
    R = x.shape[0]

    def body(x_ref, xyz_ref, W1, b1, Wsc, bsc, t_ref, sc_ref):
        xv = x_ref[...]
        f = _lrelu(_dot(xv, W1[...]) + b1[...])
        t_ref[...] = jnp.zeros((tn, w1), jnp.float32)
        t_ref[:, 0:3] = xyz_ref[...]
        t_ref[:, 3:3 + d2] = f
        sc_ref[...] = _dot(xv, Wsc[...]) + bsc[...]

    return _pallas_call(
        body, grid=(R // tn,),
        in_specs=[
            _row_spec(tn, din), _row_spec(tn, 3),
            _full_spec((din, d2)), _full_spec((1, d2)),
            _full_spec((din, dout)), _full_spec((1, dout)),
        ],
        out_specs=[_row_spec(tn, w1), _row_spec(tn, dout)],
        out_shape=[
            jax.ShapeDtypeStruct((R, w1), jnp.float32),
            jax.ShapeDtypeStruct((R, dout), jnp.float32),
        ],
    )(x, xyz, p['mlp1']['W'], p['mlp1']['b'].reshape(1, -1),
      p['shortcut']['W'], p['shortcut']['b'].reshape(1, -1))


def _rel_pos_feats(rows, xyz_c, tn, d2, Wl, bl):
    nx = rows[:, :, 0:3]
    fn = rows[:, :, 3:3 + d2]
    cx = xyz_c[:, None, :]
    rel = cx - nx
    dist = jnp.sqrt(jnp.sum(rel * rel, axis=-1, keepdims=True) + 1e-12)
    cxb = jnp.broadcast_to(cx, (tn, K_, 3))
    pe = jnp.concatenate([dist, rel, cxb, nx], axis=-1)
    f_xyz = _lrelu(
        _dot(pe.reshape(tn * K_, 10), Wl) + bl).reshape(tn, K_, d2)
    return f_xyz, fn


def _att_pool_tc(fc, Watt, Wm, bm, tn, dout, act=True):
    s = _dot(fc.reshape(tn * K_, dout), Watt).reshape(tn, K_, dout)
    m = jnp.max(s, axis=1, keepdims=True)
    e = jnp.exp(s - m)
    sm = e / jnp.sum(e, axis=1, keepdims=True)
    agg = jnp.sum(fc * sm, axis=1)
    y = _dot(agg, Wm) + bm
    return _lrelu(y) if act else y


def _block_b(rows1, xyz, p, dout, tn):
    d2 = dout // 2
    w1 = _pad16(3 + d2)
    R = xyz.shape[0]

    def body(r_ref, xyz_ref, Wl, bl, Wa, Wm, bm, t_ref):
        rows = r_ref[...].reshape(tn, K_, w1)
        xyz_c = xyz_ref[...]
        f_xyz, fn = _rel_pos_feats(rows, xyz_c, tn, d2, Wl[...], bl[...])
        fc = jnp.concatenate([f_xyz, fn], axis=-1)
        f_agg = _att_pool_tc(fc, Wa[...], Wm[...], bm[...], tn, dout)
        t_ref[...] = jnp.zeros((tn, w1), jnp.float32)
        t_ref[:, 0:3] = xyz_c
        t_ref[:, 3:3 + d2] = f_agg

    return _pallas_call(
        body, grid=(R // tn,),
        in_specs=[
            pl.BlockSpec((tn * K_, w1), lambda i: (i, 0)),
            _row_spec(tn, 3),
            _full_spec((10, d2)), _full_spec((1, d2)),
            _full_spec((dout, dout)),
            _full_spec((dout, d2)), _full_spec((1, d2)),
        ],
        out_specs=_row_spec(tn, w1),
        out_shape=jax.ShapeDtypeStruct((R, w1), jnp.float32),
    )(rows1, xyz, p['lse1']['W'], p['lse1']['b'].reshape(1, -1),
      p['att1_W'], p['att1_mlp']['W'], p['att1_mlp']['b'].reshape(1, -1))


def _block_c(rows2, xyz, sc, p, dout, tn):
    d2 = dout // 2
    w1 = _pad16(3 + d2)
    R = xyz.shape[0]

    def body(r_ref, xyz_ref, sc_ref, Wl, bl, Wl2, bl2, Wa, Wm, bm,
             W2, b2, o_ref):
        rows = r_ref[...].reshape(tn, K_, w1)
        xyz_c = xyz_ref[...]
        f_xyz, f_aggn = _rel_pos_feats(rows, xyz_c, tn, d2, Wl[...], bl[...])
        f_xyz2 = _lrelu(
            _dot(f_xyz.reshape(tn * K_, d2), Wl2[...]) + bl2[...]
        ).reshape(tn, K_, d2)
        fc2 = jnp.concatenate([f_xyz2, f_aggn], axis=-1)
        f_agg2 = _att_pool_tc(fc2, Wa[...], Wm[...], bm[...], tn, dout)
        out = _dot(f_agg2, W2[...]) + b2[...]
        o_ref[...] = _lrelu(out + sc_ref[...])

    return _pallas_call(
        body, grid=(R // tn,),
        in_specs=[
            pl.BlockSpec((tn * K_, w1), lambda i: (i, 0)),
            _row_spec(tn, 3), _row_spec(tn, dout),
            _full_spec((10, d2)), _full_spec((1, d2)),
            _full_spec((d2, d2)), _full_spec((1, d2)),
            _full_spec((dout, dout)),
            _full_spec((dout, dout)), _full_spec((1, dout)),
            _full_spec((dout, dout)), _full_spec((1, dout)),
        ],
        out_specs=_row_spec(tn, dout),
        out_shape=jax.ShapeDtypeStruct((R, dout), jnp.float32),
    )(rows2, xyz, sc,
      p['lse1']['W'], p['lse1']['b'].reshape(1, -1),
      p['lse2']['W'], p['lse2']['b'].reshape(1, -1),
      p['att2_W'], p['att2_mlp']['W'], p['att2_mlp']['b'].reshape(1, -1),
      p['mlp2']['W'], p['mlp2']['b'].reshape(1, -1))


def _max_pool_k(rows, c, tn):
    R = rows.shape[0] // K_

    def body(r_ref, o_ref):
        o_ref[...] = jnp.max(r_ref[...].reshape(tn, K_, c), axis=1)

    return _pallas_call(
        body, grid=(R // tn,),
        in_specs=[pl.BlockSpec((tn * K_, c), lambda i: (i, 0))],
        out_specs=_row_spec(tn, c),
        out_shape=jax.ShapeDtypeStruct((R, c), jnp.float32),
    )(rows)


def _decoder_step(xi, skip, W, b, tn):
    R = xi.shape[0]
    c1, c2 = xi.shape[1], skip.shape[1]
    cout = W.shape[1]
    Wa, Wb = W[:c1], W[c1:]

    def body(x_ref, s_ref, Wa_ref, Wb_ref, b_ref, o_ref):
        y = _dot(x_ref[...], Wa_ref[...]) + _dot(s_ref[...], Wb_ref[...])
        o_ref[...] = _lrelu(y + b_ref[...])

    return _pallas_call(
        body, grid=(R // tn,),
        in_specs=[
            _row_spec(tn, c1), _row_spec(tn, c2),
            _full_spec((c1, cout)), _full_spec((c2, cout)),
            _full_spec((1, cout)),
        ],
        out_specs=_row_spec(tn, cout),
        out_shape=jax.ShapeDtypeStruct((R, cout), jnp.float32),
    )(xi, skip, Wa, Wb, b.reshape(1, -1))


_TN = [1024, 512, 512, 320]


def kernel(features, xyz_0, xyz_1, xyz_2, xyz_3,
           neigh_idx_0, neigh_idx_1, neigh_idx_2, neigh_idx_3,
           sub_idx_0, sub_idx_1, sub_idx_2, sub_idx_3,
           interp_idx_0, interp_idx_1, interp_idx_2, interp_idx_3,
           params):
    xyzs = [xyz_0, xyz_1, xyz_2, xyz_3]
    neighs = [neigh_idx_0, neigh_idx_1, neigh_idx_2, neigh_idx_3]
    subs = [sub_idx_0, sub_idx_1, sub_idx_2, sub_idx_3]
    interps = [interp_idx_0, interp_idx_1, interp_idx_2, interp_idx_3]

    p = params
    enc_ch = [8] + D_OUT_

    x = _mlp_chain_kernel(
        features.reshape(B_ * NS_[0], -1),
        [p['fc']['W']], [p['fc']['b']], [True], 1024)

    skips = []
    for i in range(NUM_LAYERS_):
        n = NS_[i]
        din, dout = enc_ch[i], D_OUT_[i]
        tn = _TN[i]
        xyz = xyzs[i].reshape(B_ * n, 3)
        bp = p['blocks'][i]
        nflat = _flat_idx(neighs[i], n)

        t1, sc = _block_a(x, xyz, bp, din, dout, tn)
        rows1 = _sc_gather(t1, nflat)
        t2 = _block_b(rows1, xyz, bp, dout, tn)
        rows2 = _sc_gather(t2, nflat)
        x = _block_c(rows2, xyz, sc, bp, dout, tn)

        skips.append(x)
        sflat = _flat_idx(subs[i], n)
        srows = _sc_gather(x, sflat)
        tn_next = min(_TN[min(i + 1, 3)], B_ * NS_[i + 1])
        x = _max_pool_k(srows, dout, tn_next)

    x = _mlp_chain_kernel(
        x, [p['bottleneck']['W']], [p['bottleneck']['b']], [True], 320)

    for i in range(NUM_LAYERS_ - 1, -1, -1):
        iflat = _flat_idx(interps[i], NS_[i + 1])
        xi = _sc_gather(x, iflat)
        dp = p['decoder'][NUM_LAYERS_ - 1 - i]
        tn = 640 if B_ * NS_[i] == 1280 else 1024
        x = _decoder_step(xi, skips[i], dp['W'], dp['b'], tn)

    logits = _mlp_chain_kernel(
        x, [p['cls1']['W'], p['cls2']['W']],
        [p['cls1']['b'], p['cls2']['b']], [True, False], 1024)
    return logits.reshape(B_, NS_[0], -1)


# speedup vs baseline: 15.3521x; 2.8362x over previous
import functools

import jax
import jax.numpy as jnp
from jax import lax
from jax.experimental import pallas as pl
from jax.experimental.pallas import tpu as pltpu
from jax.experimental.pallas import tpu_sc as plsc

NUM_LAYERS_ = 4
D_OUT_ = [16, 64, 128, 256]
NS_ = [40960, 10240, 2560, 640, 160]
K_ = 16
B_ = 2

_NW = 32
_IDXROW = 128


def _pad16(c):
    return (c + 15) // 16 * 16


def _pallas_call(*args, **kwargs):
    return pl.pallas_call(*args, **kwargs)


def _lrelu(x):
    return jnp.where(x >= 0, x, 0.2 * x)


def _dot(a, b):
    return jax.lax.dot_general(
        a, b, (((a.ndim - 1,), (0,)), ((), ())),
        preferred_element_type=jnp.float32)


@functools.lru_cache(maxsize=None)
def _make_sc_gather(V, D, Mp):
    per_w = Mp // _NW
    nrow = per_w // _IDXROW
    nsub = 1
    for cand in (8, 4, 2):
        if nrow % cand == 0 and cand * _IDXROW * D * 4 <= 262144:
            nsub = cand
            break
    ch = nsub * _IDXROW
    n_iter = nrow // nsub
    mesh = plsc.VectorSubcoreMesh(core_axis_name="c", subcore_axis_name="s")

    @functools.partial(
        pl.kernel, mesh=mesh,
        compiler_params=pltpu.CompilerParams(use_tc_tiling_on_sc=False),
        out_type=jax.ShapeDtypeStruct((Mp, D), jnp.float32),
        scratch_types=[
            pltpu.VMEM((nsub, _IDXROW), jnp.int32),
            pltpu.VMEM((ch, D), jnp.float32),
            pltpu.SemaphoreType.DMA,
        ])
    def k(table_hbm, idx_hbm, out_hbm, idx_v, rows_v, sem):
        wid = lax.axis_index("s") * 2 + lax.axis_index("c")
        ibase = wid * nrow

        def body(c, carry):
            irow = ibase + c * nsub
            pltpu.sync_copy(idx_hbm.at[pl.ds(irow, nsub)], idx_v)
            cps = [
                pltpu.async_copy(table_hbm.at[idx_v.at[j]],
                                 rows_v.at[pl.ds(j * _IDXROW, _IDXROW)], sem)
                for j in range(nsub)
            ]
            for cp in cps:
                cp.wait()
            pltpu.sync_copy(rows_v, out_hbm.at[pl.ds(irow * _IDXROW, ch)])
            return carry

        lax.fori_loop(0, n_iter, body, 0)

    return k


def _sc_gather(table, idx_flat):
    V, D = table.shape
    M = idx_flat.shape[0]
    Mp = (M + 4095) // 4096 * 4096
    if Mp != M:
        idx_flat = jnp.concatenate(
            [idx_flat, jnp.zeros((Mp - M,), jnp.int32)])
    idx2d = idx_flat.reshape(Mp // _IDXROW, _IDXROW)
    out = _make_sc_gather(V, D, Mp)(table, idx2d)
    return out[:M] if Mp != M else out


def _flat_idx(idx, n_table):
    b = idx.shape[0]
    off = (jnp.arange(b, dtype=jnp.int32) * n_table)[:, None, None]
    return (idx.astype(jnp.int32) + off).reshape(-1)


def _full_spec(shape):
    return pl.BlockSpec(shape, lambda i: tuple(0 for _ in shape))


def _row_spec(tn, c):
    return pl.BlockSpec((tn, c), lambda i: (i, 0))


def _mlp_chain_kernel(x, Ws, bs, acts, tn):
    R = x.shape[0]
    cout = Ws[-1].shape[1]

    def body(x_ref, *refs):
        n = len(Ws)
        o_ref = refs[2 * n]
        y = x_ref[...]
        for t in range(n):
            y = _dot(y, refs[2 * t][...]) + refs[2 * t + 1][...]
            if acts[t]:
                y = _lrelu(y)
        o_ref[...] = y

    ins = [x]
    specs = [_row_spec(tn, x.shape[1])]
    for W, b in zip(Ws, bs):
        ins += [W, b.reshape(1, -1)]
        specs += [_full_spec(W.shape), _full_spec((1, b.shape[0]))]
    return _pallas_call(
        body, grid=(R // tn,),
        in_specs=specs,
        out_specs=_row_spec(tn, cout),
        out_shape=jax.ShapeDtypeStruct((R, cout), jnp.float32),
    )(*ins)


def _block_a(x, xyz, p, din, dout, tn):
    d2 = dout // 2
    w1 = _pad16(3 + d2)
    R = x.shape[0]

    def body(x_ref, xyz_ref, W1, b1, Wsc, bsc, t_ref, sc_ref):
        xv = x_ref[...]
        f = _lrelu(_dot(xv, W1[...]) + b1[...])
        t_ref[...] = jnp.zeros((tn, w1), jnp.float32)
        t_ref[:, 0:3] = xyz_ref[...]
        t_ref[:, 3:3 + d2] = f
        sc_ref[...] = _dot(xv, Wsc[...]) + bsc[...]

    return _pallas_call(
        body, grid=(R // tn,),
        in_specs=[
            _row_spec(tn, din), _row_spec(tn, 3),
            _full_spec((din, d2)), _full_spec((1, d2)),
            _full_spec((din, dout)), _full_spec((1, dout)),
        ],
        out_specs=[_row_spec(tn, w1), _row_spec(tn, dout)],
        out_shape=[
            jax.ShapeDtypeStruct((R, w1), jnp.float32),
            jax.ShapeDtypeStruct((R, dout), jnp.float32),
        ],
    )(x, xyz, p['mlp1']['W'], p['mlp1']['b'].reshape(1, -1),
      p['shortcut']['W'], p['shortcut']['b'].reshape(1, -1))


def _rel_pos_feats(rows, xyz_c, tn, d2, Wl, bl):
    nx = rows[:, :, 0:3]
    fn = rows[:, :, 3:3 + d2]
    cx = xyz_c[:, None, :]
    rel = cx - nx
    dist = jnp.sqrt(jnp.sum(rel * rel, axis=-1, keepdims=True) + 1e-12)
    cxb = jnp.broadcast_to(cx, (tn, K_, 3))
    pe = jnp.concatenate([dist, rel, cxb, nx], axis=-1)
    f_xyz = _lrelu(
        _dot(pe.reshape(tn * K_, 10), Wl) + bl).reshape(tn, K_, d2)
    return f_xyz, fn


def _att_pool_tc(fc, Watt, Wm, bm, tn, dout, act=True):
    s = _dot(fc.reshape(tn * K_, dout), Watt).reshape(tn, K_, dout)
    m = jnp.max(s, axis=1, keepdims=True)
    e = jnp.exp(s - m)
    sm = e / jnp.sum(e, axis=1, keepdims=True)
    agg = jnp.sum(fc * sm, axis=1)
    y = _dot(agg, Wm) + bm
    return _lrelu(y) if act else y


def _block_b(rows1, xyz, p, dout, tn):
    d2 = dout // 2
    w1 = _pad16(3 + d2)
    R = xyz.shape[0]

    def body(r_ref, xyz_ref, Wl, bl, Wa, Wm, bm, t_ref):
        rows = r_ref[...].reshape(tn, K_, w1)
        xyz_c = xyz_ref[...]
        f_xyz, fn = _rel_pos_feats(rows, xyz_c, tn, d2, Wl[...], bl[...])
        fc = jnp.concatenate([f_xyz, fn], axis=-1)
        f_agg = _att_pool_tc(fc, Wa[...], Wm[...], bm[...], tn, dout)
        t_ref[...] = jnp.zeros((tn, w1), jnp.float32)
        t_ref[:, 0:3] = xyz_c
        t_ref[:, 3:3 + d2] = f_agg

    return _pallas_call(
        body, grid=(R // tn,),
        in_specs=[
            pl.BlockSpec((tn * K_, w1), lambda i: (i, 0)),
            _row_spec(tn, 3),
            _full_spec((10, d2)), _full_spec((1, d2)),
            _full_spec((dout, dout)),
            _full_spec((dout, d2)), _full_spec((1, d2)),
        ],
        out_specs=_row_spec(tn, w1),
        out_shape=jax.ShapeDtypeStruct((R, w1), jnp.float32),
    )(rows1, xyz, p['lse1']['W'], p['lse1']['b'].reshape(1, -1),
      p['att1_W'], p['att1_mlp']['W'], p['att1_mlp']['b'].reshape(1, -1))


def _block_c(rows2, xyz, sc, p, dout, tn):
    d2 = dout // 2
    w1 = _pad16(3 + d2)
    R = xyz.shape[0]

    def body(r_ref, xyz_ref, sc_ref, Wl, bl, Wl2, bl2, Wa, Wm, bm,
             W2, b2, o_ref):
        rows = r_ref[...].reshape(tn, K_, w1)
        xyz_c = xyz_ref[...]
        f_xyz, f_aggn = _rel_pos_feats(rows, xyz_c, tn, d2, Wl[...], bl[...])
        f_xyz2 = _lrelu(
            _dot(f_xyz.reshape(tn * K_, d2), Wl2[...]) + bl2[...]
        ).reshape(tn, K_, d2)
        fc2 = jnp.concatenate([f_xyz2, f_aggn], axis=-1)
        f_agg2 = _att_pool_tc(fc2, Wa[...], Wm[...], bm[...], tn, dout)
        out = _dot(f_agg2, W2[...]) + b2[...]
        o_ref[...] = _lrelu(out + sc_ref[...])

    return _pallas_call(
        body, grid=(R // tn,),
        in_specs=[
            pl.BlockSpec((tn * K_, w1), lambda i: (i, 0)),
            _row_spec(tn, 3), _row_spec(tn, dout),
            _full_spec((10, d2)), _full_spec((1, d2)),
            _full_spec((d2, d2)), _full_spec((1, d2)),
            _full_spec((dout, dout)),
            _full_spec((dout, dout)), _full_spec((1, dout)),
            _full_spec((dout, dout)), _full_spec((1, dout)),
        ],
        out_specs=_row_spec(tn, dout),
        out_shape=jax.ShapeDtypeStruct((R, dout), jnp.float32),
    )(rows2, xyz, sc,
      p['lse1']['W'], p['lse1']['b'].reshape(1, -1),
      p['lse2']['W'], p['lse2']['b'].reshape(1, -1),
      p['att2_W'], p['att2_mlp']['W'], p['att2_mlp']['b'].reshape(1, -1),
      p['mlp2']['W'], p['mlp2']['b'].reshape(1, -1))


def _max_pool_k(rows, c, tn):
    R = rows.shape[0] // K_

    def body(r_ref, o_ref):
        o_ref[...] = jnp.max(r_ref[...].reshape(tn, K_, c), axis=1)

    return _pallas_call(
        body, grid=(R // tn,),
        in_specs=[pl.BlockSpec((tn * K_, c), lambda i: (i, 0))],
        out_specs=_row_spec(tn, c),
        out_shape=jax.ShapeDtypeStruct((R, c), jnp.float32),
    )(rows)


def _decoder_step(xi, skip, W, b, tn):
    R = xi.shape[0]
    c1, c2 = xi.shape[1], skip.shape[1]
    cout = W.shape[1]
    Wa, Wb = W[:c1], W[c1:]

    def body(x_ref, s_ref, Wa_ref, Wb_ref, b_ref, o_ref):
        y = _dot(x_ref[...], Wa_ref[...]) + _dot(s_ref[...], Wb_ref[...])
        o_ref[...] = _lrelu(y + b_ref[...])

    return _pallas_call(
        body, grid=(R // tn,),
        in_specs=[
            _row_spec(tn, c1), _row_spec(tn, c2),
            _full_spec((c1, cout)), _full_spec((c2, cout)),
            _full_spec((1, cout)),
        ],
        out_specs=_row_spec(tn, cout),
        out_shape=jax.ShapeDtypeStruct((R, cout), jnp.float32),
    )(xi, skip, Wa, Wb, b.reshape(1, -1))


_TN = [1024, 512, 512, 320]


def kernel(features, xyz_0, xyz_1, xyz_2, xyz_3,
           neigh_idx_0, neigh_idx_1, neigh_idx_2, neigh_idx_3,
           sub_idx_0, sub_idx_1, sub_idx_2, sub_idx_3,
           interp_idx_0, interp_idx_1, interp_idx_2, interp_idx_3,
           params):
    xyzs = [xyz_0, xyz_1, xyz_2, xyz_3]
    neighs = [neigh_idx_0, neigh_idx_1, neigh_idx_2, neigh_idx_3]
    subs = [sub_idx_0, sub_idx_1, sub_idx_2, sub_idx_3]
    interps = [interp_idx_0, interp_idx_1, interp_idx_2, interp_idx_3]

    p = params
    enc_ch = [8] + D_OUT_

    x = _mlp_chain_kernel(
        features.reshape(B_ * NS_[0], -1),
        [p['fc']['W']], [p['fc']['b']], [True], 1024)

    skips = []
    for i in range(NUM_LAYERS_):
        n = NS_[i]
        din, dout = enc_ch[i], D_OUT_[i]
        tn = _TN[i]
        xyz = xyzs[i].reshape(B_ * n, 3)
        bp = p['blocks'][i]
        nflat = _flat_idx(neighs[i], n)

        t1, sc = _block_a(x, xyz, bp, din, dout, tn)
        rows1 = _sc_gather(t1, nflat)
        t2 = _block_b(rows1, xyz, bp, dout, tn)
        rows2 = _sc_gather(t2, nflat)
        x = _block_c(rows2, xyz, sc, bp, dout, tn)

        skips.append(x)
        sflat = _flat_idx(subs[i], n)
        srows = _sc_gather(x, sflat)
        tn_next = min(_TN[min(i + 1, 3)], B_ * NS_[i + 1])
        x = _max_pool_k(srows, dout, tn_next)

    x = _mlp_chain_kernel(
        x, [p['bottleneck']['W']], [p['bottleneck']['b']], [True], 320)

    for i in range(NUM_LAYERS_ - 1, -1, -1):
        iflat = _flat_idx(interps[i], NS_[i + 1])
        xi = _sc_gather(x, iflat)
        dp = p['decoder'][NUM_LAYERS_ - 1 - i]
        tn = 640 if B_ * NS_[i] == 1280 else 1024
        x = _decoder_step(xi, skips[i], dp['W'], dp['b'], tn)

    logits = _mlp_chain_kernel(
        x, [p['cls1']['W'], p['cls2']['W']],
        [p['cls1']['b'], p['cls2']['b']], [True, False], 1024)
    return logits.reshape(B_, NS_[0], -1)


# speedup vs baseline: 15.7828x; 1.0281x over previous
import functools

import jax
import jax.numpy as jnp
from jax import lax
from jax.experimental import pallas as pl
from jax.experimental.pallas import tpu as pltpu
from jax.experimental.pallas import tpu_sc as plsc

NUM_LAYERS_ = 4
D_OUT_ = [16, 64, 128, 256]
NS_ = [40960, 10240, 2560, 640, 160]
K_ = 16
B_ = 2

_NW = 32
_IDXROW = 128


def _pad16(c):
    return (c + 15) // 16 * 16


def _pallas_call(*args, **kwargs):
    return pl.pallas_call(*args, **kwargs)


def _lrelu(x):
    return jnp.where(x >= 0, x, 0.2 * x)


def _dot(a, b):
    return jax.lax.dot_general(
        a, b, (((a.ndim - 1,), (0,)), ((), ())),
        preferred_element_type=jnp.float32)


@functools.lru_cache(maxsize=None)
def _make_sc_gather(V, D, Mp):
    per_w = Mp // _NW
    nrow = per_w // _IDXROW
    idx_bytes = per_w * 4
    nsub = 1
    for cand in range(min(nrow, 16), 0, -1):
        if nrow % cand == 0 and idx_bytes + 2 * cand * _IDXROW * D * 4 <= 460000:
            nsub = cand
            break
    ch = nsub * _IDXROW
    n_iter = nrow // nsub
    mesh = plsc.VectorSubcoreMesh(core_axis_name="c", subcore_axis_name="s")

    @functools.partial(
        pl.kernel, mesh=mesh,
        compiler_params=pltpu.CompilerParams(use_tc_tiling_on_sc=False),
        out_type=jax.ShapeDtypeStruct((Mp, D), jnp.float32),
        scratch_types=[
            pltpu.VMEM((per_w,), jnp.int32),
            pltpu.VMEM((2 * ch, D), jnp.float32),
            pltpu.SemaphoreType.DMA, pltpu.SemaphoreType.DMA,
            pltpu.SemaphoreType.DMA, pltpu.SemaphoreType.DMA,
        ])
    def k(table_hbm, idx_hbm, out_hbm, idx_v, rows_v, g0, g1, w0, w1):
        wid = lax.axis_index("s") * 2 + lax.axis_index("c")
        rbase = wid * per_w
        pltpu.sync_copy(idx_hbm.at[pl.ds(rbase, per_w)], idx_v)
        gsem = (g0, g1)
        wsem = (w0, w1)
        gcp = [None] * n_iter
        wcp = [None] * n_iter

        def fire(c, b):
            gcp[c] = pltpu.async_copy(
                table_hbm.at[idx_v.at[pl.ds(c * ch, ch)]],
                rows_v.at[pl.ds(b * ch, ch)], gsem[b])

        def fire_write(c, b):
            wcp[c] = pltpu.async_copy(
                rows_v.at[pl.ds(b * ch, ch)],
                out_hbm.at[pl.ds(rbase + c * ch, ch)], wsem[b])

        for c in range(n_iter):
            b = c % 2
            if c >= 2:
                wcp[c - 2].wait()
            fire(c, b)
            if c >= 1:
                gcp[c - 1].wait()
                fire_write(c - 1, 1 - b)
        gcp[n_iter - 1].wait()
        fire_write(n_iter - 1, (n_iter - 1) % 2)
        if n_iter >= 2:
            wcp[n_iter - 2].wait()
        wcp[n_iter - 1].wait()

    return k


def _sc_gather(table, idx_flat):
    V, D = table.shape
    M = idx_flat.shape[0]
    Mp = (M + 4095) // 4096 * 4096
    if Mp != M:
        idx_flat = jnp.concatenate(
            [idx_flat, jnp.zeros((Mp - M,), jnp.int32)])
    out = _make_sc_gather(V, D, Mp)(table, idx_flat)
    return out[:M] if Mp != M else out


def _flat_idx(idx, n_table):
    b = idx.shape[0]
    off = (jnp.arange(b, dtype=jnp.int32) * n_table)[:, None, None]
    return (idx.astype(jnp.int32) + off).reshape(-1)


def _full_spec(shape):
    return pl.BlockSpec(shape, lambda i: tuple(0 for _ in shape))


def _row_spec(tn, c):
    return pl.BlockSpec((tn, c), lambda i: (i, 0))


def _mlp_chain_kernel(x, Ws, bs, acts, tn):
    R = x.shape[0]
    cout = Ws[-1].shape[1]

    def body(x_ref, *refs):
        n = len(Ws)
        o_ref = refs[2 * n]
        y = x_ref[...]
        for t in range(n):
            y = _dot(y, refs[2 * t][...]) + refs[2 * t + 1][...]
            if acts[t]:
                y = _lrelu(y)
        o_ref[...] = y

    ins = [x]
    specs = [_row_spec(tn, x.shape[1])]
    for W, b in zip(Ws, bs):
        ins += [W, b.reshape(1, -1)]
        specs += [_full_spec(W.shape), _full_spec((1, b.shape[0]))]
    return _pallas_call(
        body, grid=(R // tn,),
        in_specs=specs,
        out_specs=_row_spec(tn, cout),
        out_shape=jax.ShapeDtypeStruct((R, cout), jnp.float32),
    )(*ins)


def _block_a(x, xyz, p, din, dout, tn):
    d2 = dout // 2
    w1 = _pad16(3 + d2)
    R = x.shape[0]

    def body(x_ref, xyz_ref, W1, b1, Wsc, bsc, t_ref, sc_ref):
        xv = x_ref[...]
        f = _lrelu(_dot(xv, W1[...]) + b1[...])
        t_ref[...] = jnp.zeros((tn, w1), jnp.float32)
        t_ref[:, 0:3] = xyz_ref[...]
        t_ref[:, 3:3 + d2] = f
        sc_ref[...] = _dot(xv, Wsc[...]) + bsc[...]

    return _pallas_call(
        body, grid=(R // tn,),
        in_specs=[
            _row_spec(tn, din), _row_spec(tn, 3),
            _full_spec((din, d2)), _full_spec((1, d2)),
            _full_spec((din, dout)), _full_spec((1, dout)),
        ],
        out_specs=[_row_spec(tn, w1), _row_spec(tn, dout)],
        out_shape=[
            jax.ShapeDtypeStruct((R, w1), jnp.float32),
            jax.ShapeDtypeStruct((R, dout), jnp.float32),
        ],
    )(x, xyz, p['mlp1']['W'], p['mlp1']['b'].reshape(1, -1),
      p['shortcut']['W'], p['shortcut']['b'].reshape(1, -1))


def _rel_pos_feats(rows, xyz_c, tn, d2, Wl, bl):
    nx = rows[:, :, 0:3]
    fn = rows[:, :, 3:3 + d2]
    cx = xyz_c[:, None, :]
    rel = cx - nx
    dist = jnp.sqrt(jnp.sum(rel * rel, axis=-1, keepdims=True) + 1e-12)
    cxb = jnp.broadcast_to(cx, (tn, K_, 3))
    pe = jnp.concatenate([dist, rel, cxb, nx], axis=-1)
    f_xyz = _lrelu(
        _dot(pe.reshape(tn * K_, 10), Wl) + bl).reshape(tn, K_, d2)
    return f_xyz, fn


def _att_pool_tc(fc, Watt, Wm, bm, tn, dout, act=True):
    s = _dot(fc.reshape(tn * K_, dout), Watt).reshape(tn, K_, dout)
    m = jnp.max(s, axis=1, keepdims=True)
    e = jnp.exp(s - m)
    sm = e / jnp.sum(e, axis=1, keepdims=True)
    agg = jnp.sum(fc * sm, axis=1)
    y = _dot(agg, Wm) + bm
    return _lrelu(y) if act else y


def _block_b(rows1, xyz, p, dout, tn):
    d2 = dout // 2
    w1 = _pad16(3 + d2)
    R = xyz.shape[0]

    def body(r_ref, xyz_ref, Wl, bl, Wa, Wm, bm, t_ref):
        rows = r_ref[...].reshape(tn, K_, w1)
        xyz_c = xyz_ref[...]
        f_xyz, fn = _rel_pos_feats(rows, xyz_c, tn, d2, Wl[...], bl[...])
        fc = jnp.concatenate([f_xyz, fn], axis=-1)
        f_agg = _att_pool_tc(fc, Wa[...], Wm[...], bm[...], tn, dout)
        t_ref[...] = jnp.zeros((tn, w1), jnp.float32)
        t_ref[:, 0:3] = xyz_c
        t_ref[:, 3:3 + d2] = f_agg

    return _pallas_call(
        body, grid=(R // tn,),
        in_specs=[
            pl.BlockSpec((tn * K_, w1), lambda i: (i, 0)),
            _row_spec(tn, 3),
            _full_spec((10, d2)), _full_spec((1, d2)),
            _full_spec((dout, dout)),
            _full_spec((dout, d2)), _full_spec((1, d2)),
        ],
        out_specs=_row_spec(tn, w1),
        out_shape=jax.ShapeDtypeStruct((R, w1), jnp.float32),
    )(rows1, xyz, p['lse1']['W'], p['lse1']['b'].reshape(1, -1),
      p['att1_W'], p['att1_mlp']['W'], p['att1_mlp']['b'].reshape(1, -1))


def _block_c(rows2, xyz, sc, p, dout, tn):
    d2 = dout // 2
    w1 = _pad16(3 + d2)
    R = xyz.shape[0]

    def body(r_ref, xyz_ref, sc_ref, Wl, bl, Wl2, bl2, Wa, Wm, bm,
             W2, b2, o_ref):
        rows = r_ref[...].reshape(tn, K_, w1)
        xyz_c = xyz_ref[...]
        f_xyz, f_aggn = _rel_pos_feats(rows, xyz_c, tn, d2, Wl[...], bl[...])
        f_xyz2 = _lrelu(
            _dot(f_xyz.reshape(tn * K_, d2), Wl2[...]) + bl2[...]
        ).reshape(tn, K_, d2)
        fc2 = jnp.concatenate([f_xyz2, f_aggn], axis=-1)
        f_agg2 = _att_pool_tc(fc2, Wa[...], Wm[...], bm[...], tn, dout)
        out = _dot(f_agg2, W2[...]) + b2[...]
        o_ref[...] = _lrelu(out + sc_ref[...])

    return _pallas_call(
        body, grid=(R // tn,),
        in_specs=[
            pl.BlockSpec((tn * K_, w1), lambda i: (i, 0)),
            _row_spec(tn, 3), _row_spec(tn, dout),
            _full_spec((10, d2)), _full_spec((1, d2)),
            _full_spec((d2, d2)), _full_spec((1, d2)),
            _full_spec((dout, dout)),
            _full_spec((dout, dout)), _full_spec((1, dout)),
            _full_spec((dout, dout)), _full_spec((1, dout)),
        ],
        out_specs=_row_spec(tn, dout),
        out_shape=jax.ShapeDtypeStruct((R, dout), jnp.float32),
    )(rows2, xyz, sc,
      p['lse1']['W'], p['lse1']['b'].reshape(1, -1),
      p['lse2']['W'], p['lse2']['b'].reshape(1, -1),
      p['att2_W'], p['att2_mlp']['W'], p['att2_mlp']['b'].reshape(1, -1),
      p['mlp2']['W'], p['mlp2']['b'].reshape(1, -1))


def _max_pool_k(rows, c, tn):
    R = rows.shape[0] // K_

    def body(r_ref, o_ref):
        o_ref[...] = jnp.max(r_ref[...].reshape(tn, K_, c), axis=1)

    return _pallas_call(
        body, grid=(R // tn,),
        in_specs=[pl.BlockSpec((tn * K_, c), lambda i: (i, 0))],
        out_specs=_row_spec(tn, c),
        out_shape=jax.ShapeDtypeStruct((R, c), jnp.float32),
    )(rows)


def _decoder_step(xi, skip, W, b, tn):
    R = xi.shape[0]
    c1, c2 = xi.shape[1], skip.shape[1]
    cout = W.shape[1]
    Wa, Wb = W[:c1], W[c1:]

    def body(x_ref, s_ref, Wa_ref, Wb_ref, b_ref, o_ref):
        y = _dot(x_ref[...], Wa_ref[...]) + _dot(s_ref[...], Wb_ref[...])
        o_ref[...] = _lrelu(y + b_ref[...])

    return _pallas_call(
        body, grid=(R // tn,),
        in_specs=[
            _row_spec(tn, c1), _row_spec(tn, c2),
            _full_spec((c1, cout)), _full_spec((c2, cout)),
            _full_spec((1, cout)),
        ],
        out_specs=_row_spec(tn, cout),
        out_shape=jax.ShapeDtypeStruct((R, cout), jnp.float32),
    )(xi, skip, Wa, Wb, b.reshape(1, -1))


_TN = [1024, 512, 512, 320]


def kernel(features, xyz_0, xyz_1, xyz_2, xyz_3,
           neigh_idx_0, neigh_idx_1, neigh_idx_2, neigh_idx_3,
           sub_idx_0, sub_idx_1, sub_idx_2, sub_idx_3,
           interp_idx_0, interp_idx_1, interp_idx_2, interp_idx_3,
           params):
    xyzs = [xyz_0, xyz_1, xyz_2, xyz_3]
    neighs = [neigh_idx_0, neigh_idx_1, neigh_idx_2, neigh_idx_3]
    subs = [sub_idx_0, sub_idx_1, sub_idx_2, sub_idx_3]
    interps = [interp_idx_0, interp_idx_1, interp_idx_2, interp_idx_3]

    p = params
    enc_ch = [8] + D_OUT_

    x = _mlp_chain_kernel(
        features.reshape(B_ * NS_[0], -1),
        [p['fc']['W']], [p['fc']['b']], [True], 1024)

    skips = []
    for i in range(NUM_LAYERS_):
        n = NS_[i]
        din, dout = enc_ch[i], D_OUT_[i]
        tn = _TN[i]
        xyz = xyzs[i].reshape(B_ * n, 3)
        bp = p['blocks'][i]
        nflat = _flat_idx(neighs[i], n)

        t1, sc = _block_a(x, xyz, bp, din, dout, tn)
        rows1 = _sc_gather(t1, nflat)
        t2 = _block_b(rows1, xyz, bp, dout, tn)
        rows2 = _sc_gather(t2, nflat)
        x = _block_c(rows2, xyz, sc, bp, dout, tn)

        skips.append(x)
        sflat = _flat_idx(subs[i], n)
        srows = _sc_gather(x, sflat)
        tn_next = min(_TN[min(i + 1, 3)], B_ * NS_[i + 1])
        x = _max_pool_k(srows, dout, tn_next)

    x = _mlp_chain_kernel(
        x, [p['bottleneck']['W']], [p['bottleneck']['b']], [True], 320)

    for i in range(NUM_LAYERS_ - 1, -1, -1):
        iflat = _flat_idx(interps[i], NS_[i + 1])
        xi = _sc_gather(x, iflat)
        dp = p['decoder'][NUM_LAYERS_ - 1 - i]
        tn = 640 if B_ * NS_[i] == 1280 else 1024
        x = _decoder_step(xi, skips[i], dp['W'], dp['b'], tn)

    logits = _mlp_chain_kernel(
        x, [p['cls1']['W'], p['cls2']['W']],
        [p['cls1']['b'], p['cls2']['b']], [True, False], 1024)
    return logits.reshape(B_, NS_[0], -1)


# speedup vs baseline: 27.1971x; 1.7232x over previous
import functools

import jax
import jax.numpy as jnp
from jax import lax
from jax.experimental import pallas as pl
from jax.experimental.pallas import tpu as pltpu
from jax.experimental.pallas import tpu_sc as plsc

NUM_LAYERS_ = 4
D_OUT_ = [16, 64, 128, 256]
NS_ = [40960, 10240, 2560, 640, 160]
K_ = 16
B_ = 2

_NW = 32
_IDXROW = 128


def _pad16(c):
    return (c + 15) // 16 * 16


def _pallas_call(*args, **kwargs):
    return pl.pallas_call(*args, **kwargs)


def _lrelu(x):
    return jnp.where(x >= 0, x, 0.2 * x)


def _dot(a, b):
    return jax.lax.dot_general(
        a, b, (((a.ndim - 1,), (0,)), ((), ())),
        preferred_element_type=jnp.float32)


@functools.lru_cache(maxsize=None)
def _make_sc_gather(V, D, Mp):
    per_w = Mp // _NW
    nrow = per_w // _IDXROW
    idx_bytes = per_w * 4
    nsub = 1
    for cand in range(min(nrow, 16), 0, -1):
        if nrow % cand == 0 and idx_bytes + 2 * cand * _IDXROW * D * 4 <= 460000:
            nsub = cand
            break
    ch = nsub * _IDXROW
    n_iter = nrow // nsub
    mesh = plsc.VectorSubcoreMesh(core_axis_name="c", subcore_axis_name="s")

    @functools.partial(
        pl.kernel, mesh=mesh,
        compiler_params=pltpu.CompilerParams(use_tc_tiling_on_sc=False),
        out_type=jax.ShapeDtypeStruct((Mp, D), jnp.float32),
        scratch_types=[
            pltpu.VMEM((per_w,), jnp.int32),
            pltpu.VMEM((2 * ch, D), jnp.float32),
            pltpu.SemaphoreType.DMA, pltpu.SemaphoreType.DMA,
            pltpu.SemaphoreType.DMA, pltpu.SemaphoreType.DMA,
        ])
    def k(table_hbm, idx_hbm, out_hbm, idx_v, rows_v, g0, g1, w0, w1):
        wid = lax.axis_index("s") * 2 + lax.axis_index("c")
        rbase = wid * per_w
        pltpu.sync_copy(idx_hbm.at[pl.ds(rbase, per_w)], idx_v)
        gsem = (g0, g1)
        wsem = (w0, w1)
        gcp = [None] * n_iter
        wcp = [None] * n_iter

        def fire(c, b):
            gcp[c] = pltpu.async_copy(
                table_hbm.at[idx_v.at[pl.ds(c * ch, ch)]],
                rows_v.at[pl.ds(b * ch, ch)], gsem[b])

        def fire_write(c, b):
            wcp[c] = pltpu.async_copy(
                rows_v.at[pl.ds(b * ch, ch)],
                out_hbm.at[pl.ds(rbase + c * ch, ch)], wsem[b])

        for c in range(n_iter):
            b = c % 2
            if c >= 2:
                wcp[c - 2].wait()
            fire(c, b)
            if c >= 1:
                gcp[c - 1].wait()
                fire_write(c - 1, 1 - b)
        gcp[n_iter - 1].wait()
        fire_write(n_iter - 1, (n_iter - 1) % 2)
        if n_iter >= 2:
            wcp[n_iter - 2].wait()
        wcp[n_iter - 1].wait()

    return k


def _sc_gather(table, idx_flat, wide=False):
    V, D = table.shape
    M = idx_flat.shape[0]
    Mp = (M + 4095) // 4096 * 4096
    if Mp != M:
        idx_flat = jnp.concatenate(
            [idx_flat, jnp.zeros((Mp - M,), jnp.int32)])
    out = _make_sc_gather(V, D, Mp)(table, idx_flat)
    if Mp != M:
        out = out[:M]
    return out.reshape(M // K_, D * K_) if wide else out


@functools.lru_cache(maxsize=None)
def _make_sc_gather_staged(N, D, Mp):
    per_w = Mp // _NW
    half = Mp // 2
    nrow = per_w // _IDXROW
    idx_bytes = per_w * 4
    nsub = 1
    for cand in range(min(nrow, 16), 0, -1):
        if nrow % cand == 0 and idx_bytes + 2 * cand * _IDXROW * D * 4 <= 460000:
            nsub = cand
            break
    ch = nsub * _IDXROW
    n_iter = nrow // nsub
    vs = N // 16
    mesh = plsc.VectorSubcoreMesh(core_axis_name="c", subcore_axis_name="s")

    @functools.partial(
        pl.kernel, mesh=mesh,
        compiler_params=pltpu.CompilerParams(use_tc_tiling_on_sc=False),
        out_type=jax.ShapeDtypeStruct((Mp, D), jnp.float32),
        scratch_types=[
            pltpu.VMEM((per_w,), jnp.int32),
            pltpu.VMEM((2 * ch, D), jnp.float32),
            pltpu.VMEM_SHARED((N, D), jnp.float32),
            pltpu.SemaphoreType.DMA, pltpu.SemaphoreType.DMA,
            pltpu.SemaphoreType.DMA, pltpu.SemaphoreType.DMA,
        ])
    def k(table_hbm, idx_hbm, out_hbm, idx_v, rows_v, tab_s, g0, g1, w0, w1):
        cid = lax.axis_index("c")
        sid = lax.axis_index("s")
        rbase = cid * half + sid * per_w
        pltpu.sync_copy(table_hbm.at[pl.ds(cid * N + sid * vs, vs)],
                        tab_s.at[pl.ds(sid * vs, vs)])
        pltpu.sync_copy(idx_hbm.at[pl.ds(rbase, per_w)], idx_v)
        plsc.subcore_barrier()
        gsem = (g0, g1)
        wsem = (w0, w1)
        gcp = [None] * n_iter
        wcp = [None] * n_iter

        def fire(c, b):
            gcp[c] = pltpu.async_copy(
                tab_s.at[idx_v.at[pl.ds(c * ch, ch)]],
                rows_v.at[pl.ds(b * ch, ch)], gsem[b])

        def fire_write(c, b):
            wcp[c] = pltpu.async_copy(
                rows_v.at[pl.ds(b * ch, ch)],
                out_hbm.at[pl.ds(rbase + c * ch, ch)], wsem[b])

        for c in range(n_iter):
            b = c % 2
            if c >= 2:
                wcp[c - 2].wait()
            fire(c, b)
            if c >= 1:
                gcp[c - 1].wait()
                fire_write(c - 1, 1 - b)
        gcp[n_iter - 1].wait()
        fire_write(n_iter - 1, (n_iter - 1) % 2)
        if n_iter >= 2:
            wcp[n_iter - 2].wait()
        wcp[n_iter - 1].wait()

    return k


def _sc_gather_batch_staged(table, idx, n_table):
    D = table.shape[1]
    idx_flat = idx.astype(jnp.int32).reshape(-1)
    M = idx_flat.shape[0]
    assert M % 4096 == 0 and (M // 2) % 2048 == 0
    return _make_sc_gather_staged(n_table, D, M)(table, idx_flat)


def _flat_idx(idx, n_table):
    b = idx.shape[0]
    off = (jnp.arange(b, dtype=jnp.int32) * n_table)[:, None, None]
    return (idx.astype(jnp.int32) + off).reshape(-1)


def _full_spec(shape):
    return pl.BlockSpec(shape, lambda i: tuple(0 for _ in shape))


def _row_spec(tn, c):
    return pl.BlockSpec((tn, c), lambda i: (i, 0))


def _mlp_chain_kernel(x, Ws, bs, acts, tn):
    R = x.shape[0]
    cout = Ws[-1].shape[1]

    def body(x_ref, *refs):
        n = len(Ws)
        o_ref = refs[2 * n]
        y = x_ref[...]
        for t in range(n):
            y = _dot(y, refs[2 * t][...]) + refs[2 * t + 1][...]
            if acts[t]:
                y = _lrelu(y)
        o_ref[...] = y

    ins = [x]
    specs = [_row_spec(tn, x.shape[1])]
    for W, b in zip(Ws, bs):
        ins += [W, b.reshape(1, -1)]
        specs += [_full_spec(W.shape), _full_spec((1, b.shape[0]))]
    return _pallas_call(
        body, grid=(R // tn,),
        in_specs=specs,
        out_specs=_row_spec(tn, cout),
        out_shape=jax.ShapeDtypeStruct((R, cout), jnp.float32),
    )(*ins)


def _block_a(x, xyz, p, din, dout, tn, w1=None):
    d2 = dout // 2
    if w1 is None:
        w1 = _pad16(3 + d2)
    R = x.shape[0]

    def body(x_ref, xyz_ref, W1, b1, Wsc, bsc, t_ref, sc_ref):
        xv = x_ref[...]
        f = _lrelu(_dot(xv, W1[...]) + b1[...])
        t_ref[...] = jnp.zeros((tn, w1), jnp.float32)
        t_ref[:, 0:3] = xyz_ref[...]
        t_ref[:, 3:3 + d2] = f
        sc_ref[...] = _dot(xv, Wsc[...]) + bsc[...]

    return _pallas_call(
        body, grid=(R // tn,),
        in_specs=[
            _row_spec(tn, din), _row_spec(tn, 3),
            _full_spec((din, d2)), _full_spec((1, d2)),
            _full_spec((din, dout)), _full_spec((1, dout)),
        ],
        out_specs=[_row_spec(tn, w1), _row_spec(tn, dout)],
        out_shape=[
            jax.ShapeDtypeStruct((R, w1), jnp.float32),
            jax.ShapeDtypeStruct((R, dout), jnp.float32),
        ],
    )(x, xyz, p['mlp1']['W'], p['mlp1']['b'].reshape(1, -1),
      p['shortcut']['W'], p['shortcut']['b'].reshape(1, -1))


def _rel_pos_feats(rows, xyz_c, tn, d2, Wl, bl):
    nx = rows[:, :, 0:3]
    fn = rows[:, :, 3:3 + d2]
    cx = xyz_c[:, None, :]
    rel = cx - nx
    dist = jnp.sqrt(jnp.sum(rel * rel, axis=-1, keepdims=True) + 1e-12)
    cxb = jnp.broadcast_to(cx, (tn, K_, 3))
    pe = jnp.concatenate([dist, rel, cxb, nx], axis=-1)
    f_xyz = _lrelu(
        _dot(pe.reshape(tn * K_, 10), Wl) + bl).reshape(tn, K_, d2)
    return f_xyz, fn


def _att_pool_tc(fc, Watt, Wm, bm, tn, dout, act=True):
    s = _dot(fc.reshape(tn * K_, dout), Watt).reshape(tn, K_, dout)
    m = jnp.max(s, axis=1, keepdims=True)
    e = jnp.exp(s - m)
    sm = e / jnp.sum(e, axis=1, keepdims=True)
    agg = jnp.sum(fc * sm, axis=1)
    y = _dot(agg, Wm) + bm
    return _lrelu(y) if act else y


def _block_b(rows1, xyz, p, dout, tn, w_in=None, w_out=None):
    d2 = dout // 2
    w1 = _pad16(3 + d2)
    if w_in is None:
        w_in = w1
    if w_out is None:
        w_out = w1
    R = xyz.shape[0]

    def body(r_ref, xyz_ref, Wl, bl, Wa, Wm, bm, t_ref):
        rows = r_ref[...].reshape(tn, K_, w_in)
        xyz_c = xyz_ref[...]
        f_xyz, fn = _rel_pos_feats(rows, xyz_c, tn, d2, Wl[...], bl[...])
        fc = jnp.concatenate([f_xyz, fn], axis=-1)
        f_agg = _att_pool_tc(fc, Wa[...], Wm[...], bm[...], tn, dout)
        t_ref[...] = jnp.zeros((tn, w_out), jnp.float32)
        t_ref[:, 0:3] = xyz_c
        t_ref[:, 3:3 + d2] = f_agg

    return _pallas_call(
        body, grid=(R // tn,),
        in_specs=[
            pl.BlockSpec((tn * K_, w_in), lambda i: (i, 0)),
            _row_spec(tn, 3),
            _full_spec((10, d2)), _full_spec((1, d2)),
            _full_spec((dout, dout)),
            _full_spec((dout, d2)), _full_spec((1, d2)),
        ],
        out_specs=_row_spec(tn, w_out),
        out_shape=jax.ShapeDtypeStruct((R, w_out), jnp.float32),
    )(rows1, xyz, p['lse1']['W'], p['lse1']['b'].reshape(1, -1),
      p['att1_W'], p['att1_mlp']['W'], p['att1_mlp']['b'].reshape(1, -1))


def _block_c(rows2, xyz, sc, p, dout, tn, w_in=None):
    d2 = dout // 2
    w1 = _pad16(3 + d2)
    if w_in is None:
        w_in = w1
    R = xyz.shape[0]

    def body(r_ref, xyz_ref, sc_ref, Wl, bl, Wl2, bl2, Wa, Wm, bm,
             W2, b2, o_ref):
        rows = r_ref[...].reshape(tn, K_, w_in)
        xyz_c = xyz_ref[...]
        f_xyz, f_aggn = _rel_pos_feats(rows, xyz_c, tn, d2, Wl[...], bl[...])
        f_xyz2 = _lrelu(
            _dot(f_xyz.reshape(tn * K_, d2), Wl2[...]) + bl2[...]
        ).reshape(tn, K_, d2)
        fc2 = jnp.concatenate([f_xyz2, f_aggn], axis=-1)
        f_agg2 = _att_pool_tc(fc2, Wa[...], Wm[...], bm[...], tn, dout)
        out = _dot(f_agg2, W2[...]) + b2[...]
        o_ref[...] = _lrelu(out + sc_ref[...])

    return _pallas_call(
        body, grid=(R // tn,),
        in_specs=[
            pl.BlockSpec((tn * K_, w_in), lambda i: (i, 0)),
            _row_spec(tn, 3), _row_spec(tn, dout),
            _full_spec((10, d2)), _full_spec((1, d2)),
            _full_spec((d2, d2)), _full_spec((1, d2)),
            _full_spec((dout, dout)),
            _full_spec((dout, dout)), _full_spec((1, dout)),
            _full_spec((dout, dout)), _full_spec((1, dout)),
        ],
        out_specs=_row_spec(tn, dout),
        out_shape=jax.ShapeDtypeStruct((R, dout), jnp.float32),
    )(rows2, xyz, sc,
      p['lse1']['W'], p['lse1']['b'].reshape(1, -1),
      p['lse2']['W'], p['lse2']['b'].reshape(1, -1),
      p['att2_W'], p['att2_mlp']['W'], p['att2_mlp']['b'].reshape(1, -1),
      p['mlp2']['W'], p['mlp2']['b'].reshape(1, -1))


def _max_pool_k(rows, c, tn):
    R = rows.shape[0] // K_

    def body(r_ref, o_ref):
        o_ref[...] = jnp.max(r_ref[...].reshape(tn, K_, c), axis=1)

    return _pallas_call(
        body, grid=(R // tn,),
        in_specs=[pl.BlockSpec((tn * K_, c), lambda i: (i, 0))],
        out_specs=_row_spec(tn, c),
        out_shape=jax.ShapeDtypeStruct((R, c), jnp.float32),
    )(rows)


def _decoder_step(xi, skip, W, b, tn):
    R = xi.shape[0]
    c1, c2 = xi.shape[1], skip.shape[1]
    cout = W.shape[1]
    Wa, Wb = W[:c1], W[c1:]

    def body(x_ref, s_ref, Wa_ref, Wb_ref, b_ref, o_ref):
        y = _dot(x_ref[...], Wa_ref[...]) + _dot(s_ref[...], Wb_ref[...])
        o_ref[...] = _lrelu(y + b_ref[...])

    return _pallas_call(
        body, grid=(R // tn,),
        in_specs=[
            _row_spec(tn, c1), _row_spec(tn, c2),
            _full_spec((c1, cout)), _full_spec((c2, cout)),
            _full_spec((1, cout)),
        ],
        out_specs=_row_spec(tn, cout),
        out_shape=jax.ShapeDtypeStruct((R, cout), jnp.float32),
    )(xi, skip, Wa, Wb, b.reshape(1, -1))


_W0 = 16
_D2_0 = 8
_DOUT_0 = 16


def _l0_mats(bp):
    K = K_
    eyeK = jnp.eye(K, dtype=jnp.float32)
    onesK = jnp.ones((1, K), dtype=jnp.float32)
    Wl = bp['lse1']['W']
    wd = Wl[0]
    WN = Wl[7:10] - Wl[1:4]
    WRC = Wl[1:4] + Wl[4:7]
    bl = bp['lse1']['b']
    R3 = jnp.zeros((_W0, 3), jnp.float32).at[:3, :3].set(jnp.eye(3))
    sx = jnp.zeros((_W0, 1), jnp.float32).at[:3, 0].set(1.0)
    t3 = jnp.zeros((3, _W0), jnp.float32).at[:3, :3].set(jnp.eye(3))
    PF = jnp.zeros((_W0, _DOUT_0), jnp.float32).at[
        3 + jnp.arange(8), _D2_0 + jnp.arange(8)].set(1.0)
    tc2 = jnp.zeros((_D2_0, _DOUT_0), jnp.float32).at[
        jnp.arange(8), jnp.arange(8)].set(1.0)
    wdp = jnp.concatenate([wd, jnp.zeros((8,), jnp.float32)])[None]
    m = {
        'SX': jnp.kron(eyeK, sx),
        'TCX': jnp.kron(onesK, t3),
        'BDN': jnp.kron(eyeK, jnp.concatenate(
            [R3 @ WN, jnp.zeros((_W0, 8), jnp.float32)], 1)),
        'WDT': jnp.kron(eyeK, wdp),
        'TC2': jnp.kron(onesK, tc2),
        'PFN': jnp.kron(eyeK, PF),
        'SK': jnp.kron(jnp.ones((K, 1), jnp.float32),
                       jnp.eye(_DOUT_0, dtype=jnp.float32)),
        'WRC': WRC, 'bl': bl.reshape(1, -1),
        'BDN8': jnp.kron(eyeK, R3 @ WN),
        'WDT8': jnp.kron(eyeK, wd[None]),
        'TC8': jnp.kron(onesK, jnp.eye(_D2_0, dtype=jnp.float32)),
        'BDL2': jnp.kron(eyeK, jnp.concatenate(
            [bp['lse2']['W'], jnp.zeros((8, 8), jnp.float32)], 1)),
        'bl2t': jnp.kron(onesK, jnp.concatenate(
            [bp['lse2']['b'], jnp.zeros((8,), jnp.float32)])[None]),
        'BDA': jnp.kron(eyeK, bp['att1_W']),
        'BDA2': jnp.kron(eyeK, bp['att2_W']),
    }
    return m


def _dist_wide(rw, cx, TCX, SX):
    cxw = _dot(cx, TCX)
    diff = rw - cxw
    return jnp.sqrt(_dot(diff * diff, SX) + 1e-12)


def _att_pool_wide(fcw, BDA, SK, Wm, bm):
    e = jnp.exp(_dot(fcw, BDA))
    den = _dot(e, SK)
    num = _dot(fcw * e, SK)
    return _lrelu(_dot(num / den, Wm) + bm)


def _block_b_wide(rw, t1, bp, tn):
    R = t1.shape[0]
    m = _l0_mats(bp)
    KW, KD = K_ * _W0, K_ * _DOUT_0

    def body(rw_ref, t1_ref, SX, TCX, BDN, WDT, TC2, PFN, BDA, SK,
             WRC, bl, Wm, bm, t_ref):
        rwv = rw_ref[...]
        cx = t1_ref[:, 0:3]
        dist = _dist_wide(rwv, cx, TCX[...], SX[...])
        tc = _dot(cx, WRC[...]) + bl[...]
        pre = _dot(rwv, BDN[...]) + _dot(dist, WDT[...]) + _dot(tc, TC2[...])
        fcw = _lrelu(pre) + _dot(rwv, PFN[...])
        f_agg = _att_pool_wide(fcw, BDA[...], SK[...], Wm[...], bm[...])
        t_ref[...] = jnp.zeros((tn, _W0), jnp.float32)
        t_ref[:, 0:3] = cx
        t_ref[:, 3:3 + _D2_0] = f_agg

    return _pallas_call(
        body, grid=(R // tn,),
        in_specs=[
            _row_spec(tn, KW), _row_spec(tn, _W0),
            _full_spec((KW, K_)), _full_spec((3, KW)),
            _full_spec((KW, KD)), _full_spec((K_, KD)),
            _full_spec((_D2_0, KD)), _full_spec((KW, KD)),
            _full_spec((KD, KD)), _full_spec((KD, _DOUT_0)),
            _full_spec((3, _D2_0)), _full_spec((1, _D2_0)),
            _full_spec((_DOUT_0, _D2_0)), _full_spec((1, _D2_0)),
        ],
        out_specs=_row_spec(tn, _W0),
        out_shape=jax.ShapeDtypeStruct((R, _W0), jnp.float32),
    )(rw, t1, m['SX'], m['TCX'], m['BDN'], m['WDT'], m['TC2'], m['PFN'],
      m['BDA'], m['SK'], m['WRC'], m['bl'],
      bp['att1_mlp']['W'], bp['att1_mlp']['b'].reshape(1, -1))


def _block_c_wide(rw, t2, sc, bp, tn):
    R = t2.shape[0]
    m = _l0_mats(bp)
    KW, KD, K8 = K_ * _W0, K_ * _DOUT_0, K_ * _D2_0

    def body(rw_ref, t2_ref, sc_ref, SX, TCX, BDN8, WDT8, TC8, BDL2, bl2t,
             PFN, BDA2, SK, WRC, bl, Wm, bm, W2, b2, o_ref):
        rwv = rw_ref[...]
        cx = t2_ref[:, 0:3]
        dist = _dist_wide(rwv, cx, TCX[...], SX[...])
        tc = _dot(cx, WRC[...]) + bl[...]
        f_xyz = _lrelu(_dot(rwv, BDN8[...]) + _dot(dist, WDT8[...])
                       + _dot(tc, TC8[...]))
        fc2 = _lrelu(_dot(f_xyz, BDL2[...]) + bl2t[...]) + _dot(rwv, PFN[...])
        f_agg2 = _att_pool_wide(fc2, BDA2[...], SK[...], Wm[...], bm[...])
        out = _dot(f_agg2, W2[...]) + b2[...]
        o_ref[...] = _lrelu(out + sc_ref[...])

    return _pallas_call(
        body, grid=(R // tn,),
        in_specs=[
            _row_spec(tn, KW), _row_spec(tn, _W0), _row_spec(tn, _DOUT_0),
            _full_spec((KW, K_)), _full_spec((3, KW)),
            _full_spec((KW, K8)), _full_spec((K_, K8)),
            _full_spec((_D2_0, K8)), _full_spec((K8, KD)),
            _full_spec((1, KD)), _full_spec((KW, KD)),
            _full_spec((KD, KD)), _full_spec((KD, _DOUT_0)),
            _full_spec((3, _D2_0)), _full_spec((1, _D2_0)),
            _full_spec((_DOUT_0, _DOUT_0)), _full_spec((1, _DOUT_0)),
            _full_spec((_DOUT_0, _DOUT_0)), _full_spec((1, _DOUT_0)),
        ],
        out_specs=_row_spec(tn, _DOUT_0),
        out_shape=jax.ShapeDtypeStruct((R, _DOUT_0), jnp.float32),
    )(rw, t2, sc, m['SX'], m['TCX'], m['BDN8'], m['WDT8'], m['TC8'],
      m['BDL2'], m['bl2t'], m['PFN'], m['BDA2'], m['SK'], m['WRC'], m['bl'],
      bp['att2_mlp']['W'], bp['att2_mlp']['b'].reshape(1, -1),
      bp['mlp2']['W'], bp['mlp2']['b'].reshape(1, -1))


_TN = [1024, 512, 512, 320]


def kernel(features, xyz_0, xyz_1, xyz_2, xyz_3,
           neigh_idx_0, neigh_idx_1, neigh_idx_2, neigh_idx_3,
           sub_idx_0, sub_idx_1, sub_idx_2, sub_idx_3,
           interp_idx_0, interp_idx_1, interp_idx_2, interp_idx_3,
           params):
    xyzs = [xyz_0, xyz_1, xyz_2, xyz_3]
    neighs = [neigh_idx_0, neigh_idx_1, neigh_idx_2, neigh_idx_3]
    subs = [sub_idx_0, sub_idx_1, sub_idx_2, sub_idx_3]
    interps = [interp_idx_0, interp_idx_1, interp_idx_2, interp_idx_3]

    p = params
    enc_ch = [8] + D_OUT_

    x = _mlp_chain_kernel(
        features.reshape(B_ * NS_[0], -1),
        [p['fc']['W']], [p['fc']['b']], [True], 1024)

    skips = []
    for i in range(NUM_LAYERS_):
        n = NS_[i]
        din, dout = enc_ch[i], D_OUT_[i]
        tn = _TN[i]
        xyz = xyzs[i].reshape(B_ * n, 3)
        bp = p['blocks'][i]
        nflat = _flat_idx(neighs[i], n)

        t1, sc = _block_a(x, xyz, bp, din, dout, tn)
        if i == 0:
            rows1w = _sc_gather(t1, nflat, wide=True)
            t2 = _block_b_wide(rows1w, t1, bp, tn)
            rows2w = _sc_gather(t2, nflat, wide=True)
            x = _block_c_wide(rows2w, t2, sc, bp, tn)
        else:
            rows1 = _sc_gather(t1, nflat)
            t2 = _block_b(rows1, xyz, bp, dout, tn)
            rows2 = _sc_gather(t2, nflat)
            x = _block_c(rows2, xyz, sc, bp, dout, tn)

        skips.append(x)
        sflat = _flat_idx(subs[i], n)
        srows = _sc_gather(x, sflat)
        tn_next = min(_TN[min(i + 1, 3)], B_ * NS_[i + 1])
        x = _max_pool_k(srows, dout, tn_next)

    x = _mlp_chain_kernel(
        x, [p['bottleneck']['W']], [p['bottleneck']['b']], [True], 320)

    for i in range(NUM_LAYERS_ - 1, -1, -1):
        iflat = _flat_idx(interps[i], NS_[i + 1])
        if i == 3:
            x = jnp.concatenate(
                [x, jnp.zeros((1280 - x.shape[0], x.shape[1]), x.dtype)])
            iflat = jnp.concatenate(
                [iflat, jnp.zeros((8192 - iflat.shape[0],), jnp.int32)])
            xi = _sc_gather(x, iflat)[:B_ * NS_[3]]
        else:
            xi = _sc_gather(x, iflat)
        dp = p['decoder'][NUM_LAYERS_ - 1 - i]
        tn = 640 if B_ * NS_[i] == 1280 else 1024
        x = _decoder_step(xi, skips[i], dp['W'], dp['b'], tn)

    logits = _mlp_chain_kernel(
        x, [p['cls1']['W'], p['cls2']['W']],
        [p['cls1']['b'], p['cls2']['b']], [True, False], 1024)
    return logits.reshape(B_, NS_[0], -1)


# speedup vs baseline: 27.7955x; 1.0220x over previous
import functools

import jax
import jax.numpy as jnp
from jax import lax
from jax.experimental import pallas as pl
from jax.experimental.pallas import tpu as pltpu
from jax.experimental.pallas import tpu_sc as plsc

NUM_LAYERS_ = 4
D_OUT_ = [16, 64, 128, 256]
NS_ = [40960, 10240, 2560, 640, 160]
K_ = 16
B_ = 2

_NW = 32
_IDXROW = 128


def _pad16(c):
    return (c + 15) // 16 * 16


def _pallas_call(*args, **kwargs):
    return pl.pallas_call(*args, **kwargs)


def _lrelu(x):
    return jnp.where(x >= 0, x, 0.2 * x)


def _dot(a, b):
    return jax.lax.dot_general(
        a, b, (((a.ndim - 1,), (0,)), ((), ())),
        preferred_element_type=jnp.float32)


@functools.lru_cache(maxsize=None)
def _make_sc_gather(V, D, Mp):
    per_w = Mp // _NW
    nrow = per_w // _IDXROW
    idx_bytes = per_w * 4
    nsub = 1
    for cand in range(min(nrow, 16), 0, -1):
        if nrow % cand == 0 and idx_bytes + 2 * cand * _IDXROW * D * 4 <= 460000:
            nsub = cand
            break
    ch = nsub * _IDXROW
    n_iter = nrow // nsub
    mesh = plsc.VectorSubcoreMesh(core_axis_name="c", subcore_axis_name="s")

    @functools.partial(
        pl.kernel, mesh=mesh,
        compiler_params=pltpu.CompilerParams(use_tc_tiling_on_sc=False),
        out_type=jax.ShapeDtypeStruct((Mp, D), jnp.float32),
        scratch_types=[
            pltpu.VMEM((per_w,), jnp.int32),
            pltpu.VMEM((2 * ch, D), jnp.float32),
            pltpu.SemaphoreType.DMA, pltpu.SemaphoreType.DMA,
            pltpu.SemaphoreType.DMA, pltpu.SemaphoreType.DMA,
        ])
    def k(table_hbm, idx_hbm, out_hbm, idx_v, rows_v, g0, g1, w0, w1):
        wid = lax.axis_index("s") * 2 + lax.axis_index("c")
        rbase = wid * per_w
        pltpu.sync_copy(idx_hbm.at[pl.ds(rbase, per_w)], idx_v)
        gsem = (g0, g1)
        wsem = (w0, w1)
        gcp = [None] * n_iter
        wcp = [None] * n_iter

        def fire(c, b):
            gcp[c] = pltpu.async_copy(
                table_hbm.at[idx_v.at[pl.ds(c * ch, ch)]],
                rows_v.at[pl.ds(b * ch, ch)], gsem[b])

        def fire_write(c, b):
            wcp[c] = pltpu.async_copy(
                rows_v.at[pl.ds(b * ch, ch)],
                out_hbm.at[pl.ds(rbase + c * ch, ch)], wsem[b])

        for c in range(n_iter):
            b = c % 2
            if c >= 2:
                wcp[c - 2].wait()
            fire(c, b)
            if c >= 1:
                gcp[c - 1].wait()
                fire_write(c - 1, 1 - b)
        gcp[n_iter - 1].wait()
        fire_write(n_iter - 1, (n_iter - 1) % 2)
        if n_iter >= 2:
            wcp[n_iter - 2].wait()
        wcp[n_iter - 1].wait()

    return k


def _sc_gather(table, idx_flat, wide=False):
    V, D = table.shape
    M = idx_flat.shape[0]
    Mp = (M + 4095) // 4096 * 4096
    if Mp != M:
        idx_flat = jnp.concatenate(
            [idx_flat, jnp.zeros((Mp - M,), jnp.int32)])
    out = _make_sc_gather(V, D, Mp)(table, idx_flat)
    if Mp != M:
        out = out[:M]
    return out.reshape(M // K_, D * K_) if wide else out


def _flat_idx(idx, n_table):
    b = idx.shape[0]
    off = (jnp.arange(b, dtype=jnp.int32) * n_table)[:, None, None]
    return (idx.astype(jnp.int32) + off).reshape(-1)


def _full_spec(shape):
    return pl.BlockSpec(shape, lambda i: tuple(0 for _ in shape))


def _row_spec(tn, c):
    return pl.BlockSpec((tn, c), lambda i: (i, 0))


def _mlp_chain_kernel(x, Ws, bs, acts, tn):
    R = x.shape[0]
    cout = Ws[-1].shape[1]

    def body(x_ref, *refs):
        n = len(Ws)
        o_ref = refs[2 * n]
        y = x_ref[...]
        for t in range(n):
            y = _dot(y, refs[2 * t][...]) + refs[2 * t + 1][...]
            if acts[t]:
                y = _lrelu(y)
        o_ref[...] = y

    ins = [x]
    specs = [_row_spec(tn, x.shape[1])]
    for W, b in zip(Ws, bs):
        ins += [W, b.reshape(1, -1)]
        specs += [_full_spec(W.shape), _full_spec((1, b.shape[0]))]
    return _pallas_call(
        body, grid=(R // tn,),
        in_specs=specs,
        out_specs=_row_spec(tn, cout),
        out_shape=jax.ShapeDtypeStruct((R, cout), jnp.float32),
    )(*ins)


def _block_a(x, xyz, p, din, dout, tn, w1=None):
    d2 = dout // 2
    if w1 is None:
        w1 = _pad16(3 + d2)
    R = x.shape[0]

    def body(x_ref, xyz_ref, W1, b1, Wsc, bsc, t_ref, sc_ref):
        xv = x_ref[...]
        f = _lrelu(_dot(xv, W1[...]) + b1[...])
        t_ref[...] = jnp.zeros((tn, w1), jnp.float32)
        t_ref[:, 0:3] = xyz_ref[...]
        t_ref[:, 3:3 + d2] = f
        sc_ref[...] = _dot(xv, Wsc[...]) + bsc[...]

    return _pallas_call(
        body, grid=(R // tn,),
        in_specs=[
            _row_spec(tn, din), _row_spec(tn, 3),
            _full_spec((din, d2)), _full_spec((1, d2)),
            _full_spec((din, dout)), _full_spec((1, dout)),
        ],
        out_specs=[_row_spec(tn, w1), _row_spec(tn, dout)],
        out_shape=[
            jax.ShapeDtypeStruct((R, w1), jnp.float32),
            jax.ShapeDtypeStruct((R, dout), jnp.float32),
        ],
    )(x, xyz, p['mlp1']['W'], p['mlp1']['b'].reshape(1, -1),
      p['shortcut']['W'], p['shortcut']['b'].reshape(1, -1))


def _rel_pos_feats(rows, xyz_c, tn, d2, Wl, bl):
    nx = rows[:, :, 0:3]
    fn = rows[:, :, 3:3 + d2]
    cx = xyz_c[:, None, :]
    rel = cx - nx
    dist = jnp.sqrt(jnp.sum(rel * rel, axis=-1, keepdims=True) + 1e-12)
    cxb = jnp.broadcast_to(cx, (tn, K_, 3))
    pe = jnp.concatenate([dist, rel, cxb, nx], axis=-1)
    f_xyz = _lrelu(
        _dot(pe.reshape(tn * K_, 10), Wl) + bl).reshape(tn, K_, d2)
    return f_xyz, fn


def _att_pool_tc(fc, Watt, Wm, bm, tn, dout, act=True):
    s = _dot(fc.reshape(tn * K_, dout), Watt).reshape(tn, K_, dout)
    e = jnp.exp(s)
    num = jnp.sum(fc * e, axis=1)
    den = jnp.sum(e, axis=1)
    y = _dot(num / den, Wm) + bm
    return _lrelu(y) if act else y


def _block_b(rows1, xyz, p, dout, tn, w_in=None, w_out=None):
    d2 = dout // 2
    w1 = _pad16(3 + d2)
    if w_in is None:
        w_in = w1
    if w_out is None:
        w_out = w1
    R = xyz.shape[0]

    def body(r_ref, xyz_ref, Wl, bl, Wa, Wm, bm, t_ref):
        rows = r_ref[...].reshape(tn, K_, w_in)
        xyz_c = xyz_ref[...]
        f_xyz, fn = _rel_pos_feats(rows, xyz_c, tn, d2, Wl[...], bl[...])
        fc = jnp.concatenate([f_xyz, fn], axis=-1)
        f_agg = _att_pool_tc(fc, Wa[...], Wm[...], bm[...], tn, dout)
        t_ref[...] = jnp.zeros((tn, w_out), jnp.float32)
        t_ref[:, 0:3] = xyz_c
        t_ref[:, 3:3 + d2] = f_agg

    return _pallas_call(
        body, grid=(R // tn,),
        in_specs=[
            pl.BlockSpec((tn * K_, w_in), lambda i: (i, 0)),
            _row_spec(tn, 3),
            _full_spec((10, d2)), _full_spec((1, d2)),
            _full_spec((dout, dout)),
            _full_spec((dout, d2)), _full_spec((1, d2)),
        ],
        out_specs=_row_spec(tn, w_out),
        out_shape=jax.ShapeDtypeStruct((R, w_out), jnp.float32),
    )(rows1, xyz, p['lse1']['W'], p['lse1']['b'].reshape(1, -1),
      p['att1_W'], p['att1_mlp']['W'], p['att1_mlp']['b'].reshape(1, -1))


def _block_c(rows2, xyz, sc, p, dout, tn, w_in=None):
    d2 = dout // 2
    w1 = _pad16(3 + d2)
    if w_in is None:
        w_in = w1
    R = xyz.shape[0]

    def body(r_ref, xyz_ref, sc_ref, Wl, bl, Wl2, bl2, Wa, Wm, bm,
             W2, b2, o_ref):
        rows = r_ref[...].reshape(tn, K_, w_in)
        xyz_c = xyz_ref[...]
        f_xyz, f_aggn = _rel_pos_feats(rows, xyz_c, tn, d2, Wl[...], bl[...])
        f_xyz2 = _lrelu(
            _dot(f_xyz.reshape(tn * K_, d2), Wl2[...]) + bl2[...]
        ).reshape(tn, K_, d2)
        fc2 = jnp.concatenate([f_xyz2, f_aggn], axis=-1)
        f_agg2 = _att_pool_tc(fc2, Wa[...], Wm[...], bm[...], tn, dout)
        out = _dot(f_agg2, W2[...]) + b2[...]
        o_ref[...] = _lrelu(out + sc_ref[...])

    return _pallas_call(
        body, grid=(R // tn,),
        in_specs=[
            pl.BlockSpec((tn * K_, w_in), lambda i: (i, 0)),
            _row_spec(tn, 3), _row_spec(tn, dout),
            _full_spec((10, d2)), _full_spec((1, d2)),
            _full_spec((d2, d2)), _full_spec((1, d2)),
            _full_spec((dout, dout)),
            _full_spec((dout, dout)), _full_spec((1, dout)),
            _full_spec((dout, dout)), _full_spec((1, dout)),
        ],
        out_specs=_row_spec(tn, dout),
        out_shape=jax.ShapeDtypeStruct((R, dout), jnp.float32),
    )(rows2, xyz, sc,
      p['lse1']['W'], p['lse1']['b'].reshape(1, -1),
      p['lse2']['W'], p['lse2']['b'].reshape(1, -1),
      p['att2_W'], p['att2_mlp']['W'], p['att2_mlp']['b'].reshape(1, -1),
      p['mlp2']['W'], p['mlp2']['b'].reshape(1, -1))


def _max_pool_k(rows, c, tn):
    R = rows.shape[0] // K_

    def body(r_ref, o_ref):
        o_ref[...] = jnp.max(r_ref[...].reshape(tn, K_, c), axis=1)

    return _pallas_call(
        body, grid=(R // tn,),
        in_specs=[pl.BlockSpec((tn * K_, c), lambda i: (i, 0))],
        out_specs=_row_spec(tn, c),
        out_shape=jax.ShapeDtypeStruct((R, c), jnp.float32),
    )(rows)


def _decoder_step(xi, skip, W, b, tn):
    R = xi.shape[0]
    c1, c2 = xi.shape[1], skip.shape[1]
    cout = W.shape[1]
    Wa, Wb = W[:c1], W[c1:]

    def body(x_ref, s_ref, Wa_ref, Wb_ref, b_ref, o_ref):
        y = _dot(x_ref[...], Wa_ref[...]) + _dot(s_ref[...], Wb_ref[...])
        o_ref[...] = _lrelu(y + b_ref[...])

    return _pallas_call(
        body, grid=(R // tn,),
        in_specs=[
            _row_spec(tn, c1), _row_spec(tn, c2),
            _full_spec((c1, cout)), _full_spec((c2, cout)),
            _full_spec((1, cout)),
        ],
        out_specs=_row_spec(tn, cout),
        out_shape=jax.ShapeDtypeStruct((R, cout), jnp.float32),
    )(xi, skip, Wa, Wb, b.reshape(1, -1))


_W0 = 16
_D2_0 = 8
_DOUT_0 = 16


def _l0_mats(bp):
    K = K_
    eyeK = jnp.eye(K, dtype=jnp.float32)
    onesK = jnp.ones((1, K), dtype=jnp.float32)
    Wl = bp['lse1']['W']
    wd = Wl[0]
    WN = Wl[7:10] - Wl[1:4]
    WRC = Wl[1:4] + Wl[4:7]
    bl = bp['lse1']['b']
    R3 = jnp.zeros((_W0, 3), jnp.float32).at[:3, :3].set(jnp.eye(3))
    sx = jnp.zeros((_W0, 1), jnp.float32).at[:3, 0].set(1.0)
    t3 = jnp.zeros((3, _W0), jnp.float32).at[:3, :3].set(jnp.eye(3))
    PF = jnp.zeros((_W0, _DOUT_0), jnp.float32).at[
        3 + jnp.arange(8), _D2_0 + jnp.arange(8)].set(1.0)
    tc2 = jnp.zeros((_D2_0, _DOUT_0), jnp.float32).at[
        jnp.arange(8), jnp.arange(8)].set(1.0)
    wdp = jnp.concatenate([wd, jnp.zeros((8,), jnp.float32)])[None]
    m = {
        'SX': jnp.kron(eyeK, sx),
        'TCX': jnp.kron(onesK, t3),
        'BDN': jnp.kron(eyeK, jnp.concatenate(
            [R3 @ WN, jnp.zeros((_W0, 8), jnp.float32)], 1)),
        'WDT': jnp.kron(eyeK, wdp),
        'TC2': jnp.kron(onesK, tc2),
        'PFN': jnp.kron(eyeK, PF),
        'SK': jnp.kron(jnp.ones((K, 1), jnp.float32),
                       jnp.eye(_DOUT_0, dtype=jnp.float32)),
        'WRC': WRC, 'bl': bl.reshape(1, -1),
        'BDN8': jnp.kron(eyeK, R3 @ WN),
        'WDT8': jnp.kron(eyeK, wd[None]),
        'TC8': jnp.kron(onesK, jnp.eye(_D2_0, dtype=jnp.float32)),
        'BDL2': jnp.kron(eyeK, jnp.concatenate(
            [bp['lse2']['W'], jnp.zeros((8, 8), jnp.float32)], 1)),
        'bl2t': jnp.kron(onesK, jnp.concatenate(
            [bp['lse2']['b'], jnp.zeros((8,), jnp.float32)])[None]),
        'BDA': jnp.kron(eyeK, bp['att1_W']),
        'BDA2': jnp.kron(eyeK, bp['att2_W']),
    }
    return m


def _dist_wide(rw, cx, TCX, SX):
    cxw = _dot(cx, TCX)
    diff = rw - cxw
    return jnp.sqrt(_dot(diff * diff, SX) + 1e-12)


def _att_pool_wide(fcw, BDA, SK, Wm, bm):
    e = jnp.exp(_dot(fcw, BDA))
    den = _dot(e, SK)
    num = _dot(fcw * e, SK)
    return _lrelu(_dot(num / den, Wm) + bm)


def _block_b_wide(rw, t1, bp, tn):
    R = t1.shape[0]
    m = _l0_mats(bp)
    KW, KD = K_ * _W0, K_ * _DOUT_0

    def body(rw_ref, t1_ref, SX, TCX, BDN, WDT, TC2, PFN, BDA, SK,
             WRC, bl, Wm, bm, t_ref):
        rwv = rw_ref[...]
        cx = t1_ref[:, 0:3]
        dist = _dist_wide(rwv, cx, TCX[...], SX[...])
        tc = _dot(cx, WRC[...]) + bl[...]
        pre = _dot(rwv, BDN[...]) + _dot(dist, WDT[...]) + _dot(tc, TC2[...])
        fcw = _lrelu(pre) + _dot(rwv, PFN[...])
        f_agg = _att_pool_wide(fcw, BDA[...], SK[...], Wm[...], bm[...])
        t_ref[...] = jnp.zeros((tn, _W0), jnp.float32)
        t_ref[:, 0:3] = cx
        t_ref[:, 3:3 + _D2_0] = f_agg

    return _pallas_call(
        body, grid=(R // tn,),
        in_specs=[
            _row_spec(tn, KW), _row_spec(tn, _W0),
            _full_spec((KW, K_)), _full_spec((3, KW)),
            _full_spec((KW, KD)), _full_spec((K_, KD)),
            _full_spec((_D2_0, KD)), _full_spec((KW, KD)),
            _full_spec((KD, KD)), _full_spec((KD, _DOUT_0)),
            _full_spec((3, _D2_0)), _full_spec((1, _D2_0)),
            _full_spec((_DOUT_0, _D2_0)), _full_spec((1, _D2_0)),
        ],
        out_specs=_row_spec(tn, _W0),
        out_shape=jax.ShapeDtypeStruct((R, _W0), jnp.float32),
    )(rw, t1, m['SX'], m['TCX'], m['BDN'], m['WDT'], m['TC2'], m['PFN'],
      m['BDA'], m['SK'], m['WRC'], m['bl'],
      bp['att1_mlp']['W'], bp['att1_mlp']['b'].reshape(1, -1))


def _block_c_wide(rw, t2, sc, bp, tn):
    R = t2.shape[0]
    m = _l0_mats(bp)
    KW, KD, K8 = K_ * _W0, K_ * _DOUT_0, K_ * _D2_0

    def body(rw_ref, t2_ref, sc_ref, SX, TCX, BDN8, WDT8, TC8, BDL2, bl2t,
             PFN, BDA2, SK, WRC, bl, Wm, bm, W2, b2, o_ref):
        rwv = rw_ref[...]
        cx = t2_ref[:, 0:3]
        dist = _dist_wide(rwv, cx, TCX[...], SX[...])
        tc = _dot(cx, WRC[...]) + bl[...]
        f_xyz = _lrelu(_dot(rwv, BDN8[...]) + _dot(dist, WDT8[...])
                       + _dot(tc, TC8[...]))
        fc2 = _lrelu(_dot(f_xyz, BDL2[...]) + bl2t[...]) + _dot(rwv, PFN[...])
        f_agg2 = _att_pool_wide(fc2, BDA2[...], SK[...], Wm[...], bm[...])
        out = _dot(f_agg2, W2[...]) + b2[...]
        o_ref[...] = _lrelu(out + sc_ref[...])

    return _pallas_call(
        body, grid=(R // tn,),
        in_specs=[
            _row_spec(tn, KW), _row_spec(tn, _W0), _row_spec(tn, _DOUT_0),
            _full_spec((KW, K_)), _full_spec((3, KW)),
            _full_spec((KW, K8)), _full_spec((K_, K8)),
            _full_spec((_D2_0, K8)), _full_spec((K8, KD)),
            _full_spec((1, KD)), _full_spec((KW, KD)),
            _full_spec((KD, KD)), _full_spec((KD, _DOUT_0)),
            _full_spec((3, _D2_0)), _full_spec((1, _D2_0)),
            _full_spec((_DOUT_0, _DOUT_0)), _full_spec((1, _DOUT_0)),
            _full_spec((_DOUT_0, _DOUT_0)), _full_spec((1, _DOUT_0)),
        ],
        out_specs=_row_spec(tn, _DOUT_0),
        out_shape=jax.ShapeDtypeStruct((R, _DOUT_0), jnp.float32),
    )(rw, t2, sc, m['SX'], m['TCX'], m['BDN8'], m['WDT8'], m['TC8'],
      m['BDL2'], m['bl2t'], m['PFN'], m['BDA2'], m['SK'], m['WRC'], m['bl'],
      bp['att2_mlp']['W'], bp['att2_mlp']['b'].reshape(1, -1),
      bp['mlp2']['W'], bp['mlp2']['b'].reshape(1, -1))


_TN = [1024, 512, 512, 320]


def kernel(features, xyz_0, xyz_1, xyz_2, xyz_3,
           neigh_idx_0, neigh_idx_1, neigh_idx_2, neigh_idx_3,
           sub_idx_0, sub_idx_1, sub_idx_2, sub_idx_3,
           interp_idx_0, interp_idx_1, interp_idx_2, interp_idx_3,
           params):
    xyzs = [xyz_0, xyz_1, xyz_2, xyz_3]
    neighs = [neigh_idx_0, neigh_idx_1, neigh_idx_2, neigh_idx_3]
    subs = [sub_idx_0, sub_idx_1, sub_idx_2, sub_idx_3]
    interps = [interp_idx_0, interp_idx_1, interp_idx_2, interp_idx_3]

    p = params
    enc_ch = [8] + D_OUT_

    x = _mlp_chain_kernel(
        features.reshape(B_ * NS_[0], -1),
        [p['fc']['W']], [p['fc']['b']], [True], 1024)

    skips = []
    for i in range(NUM_LAYERS_):
        n = NS_[i]
        din, dout = enc_ch[i], D_OUT_[i]
        tn = _TN[i]
        xyz = xyzs[i].reshape(B_ * n, 3)
        bp = p['blocks'][i]
        nflat = _flat_idx(neighs[i], n)

        t1, sc = _block_a(x, xyz, bp, din, dout, tn)
        if i == 0:
            rows1w = _sc_gather(t1, nflat, wide=True)
            t2 = _block_b_wide(rows1w, t1, bp, tn)
            rows2w = _sc_gather(t2, nflat, wide=True)
            x = _block_c_wide(rows2w, t2, sc, bp, tn)
        else:
            rows1 = _sc_gather(t1, nflat)
            t2 = _block_b(rows1, xyz, bp, dout, tn)
            rows2 = _sc_gather(t2, nflat)
            x = _block_c(rows2, xyz, sc, bp, dout, tn)

        skips.append(x)
        sflat = _flat_idx(subs[i], n)
        srows = _sc_gather(x, sflat)
        tn_next = min(_TN[min(i + 1, 3)], B_ * NS_[i + 1])
        x = _max_pool_k(srows, dout, tn_next)

    x = _mlp_chain_kernel(
        x, [p['bottleneck']['W']], [p['bottleneck']['b']], [True], 320)

    for i in range(NUM_LAYERS_ - 1, -1, -1):
        iflat = _flat_idx(interps[i], NS_[i + 1])
        if i == 3:
            x = jnp.concatenate(
                [x, jnp.zeros((1280 - x.shape[0], x.shape[1]), x.dtype)])
            iflat = jnp.concatenate(
                [iflat, jnp.zeros((8192 - iflat.shape[0],), jnp.int32)])
            xi = _sc_gather(x, iflat)[:B_ * NS_[3]]
        else:
            xi = _sc_gather(x, iflat)
        dp = p['decoder'][NUM_LAYERS_ - 1 - i]
        tn = 640 if B_ * NS_[i] == 1280 else 1024
        x = _decoder_step(xi, skips[i], dp['W'], dp['b'], tn)

    logits = _mlp_chain_kernel(
        x, [p['cls1']['W'], p['cls2']['W']],
        [p['cls1']['b'], p['cls2']['b']], [True, False], 1024)
    return logits.reshape(B_, NS_[0], -1)
